# baseline (speedup 1.0000x reference)
import functools

import numpy as np
import jax
import jax.numpy as jnp
from jax.experimental import pallas as pl
from jax.experimental.pallas import tpu as pltpu

_B, _T, _C, _V = 2, 2048, 512, 32000
_N = 16
_H = _C // _N
_L = 32
_TP = 256
_TC = 256
_TR = 256
_VT = 3200
_G = 8

_F32 = jnp.float32


def _hmask():
    return jnp.asarray(np.kron(np.eye(_H, dtype=np.float32),
                               np.ones((_N, _N), np.float32)))


def _masks():
    tri = np.tril(np.ones((_L, _L), np.float32))
    strict = np.tril(np.ones((_L, _L), np.float32), -1)
    eye = np.eye(_L, dtype=np.float32)
    return jnp.asarray(np.stack([tri, strict, eye]))


def _dot(a, b, prec=None):
    return jnp.dot(a, b, preferred_element_type=_F32, precision=prec)


def _bdot(a, b, ca, cb, prec=None):
    return jax.lax.dot_general(
        a, b, (((ca,), (cb,)), ((0,), (0,))),
        preferred_element_type=_F32, precision=prec)


_BF16 = jnp.bfloat16


def _bf(x):
    return x.astype(_BF16)


def _ln(x, g, b, eps=1e-5):
    m = jnp.mean(x, axis=-1, keepdims=True)
    c = x - m
    v = jnp.mean(c * c, axis=-1, keepdims=True)
    return c * jax.lax.rsqrt(v + eps) * g + b


def _embed_kernel(tok_ref, *refs):
    o_ref = refs[_G]
    for g in range(_G):
        o_ref[0, 0, g, :] = refs[g][0, 0, :]


def _embed(tokens, emb):
    tok = tokens.reshape(-1)
    n = tok.shape[0] // _G
    n2 = n // 2
    emb3 = emb.reshape(_V, 1, _C)

    def _imap(g, c, i, tr):
        return (tr[(c * n2 + i) * _G + g], 0, 0)

    in_specs = [pl.BlockSpec((1, 1, _C), functools.partial(_imap, g))
                for g in range(_G)]
    out = pl.pallas_call(
        _embed_kernel,
        grid_spec=pltpu.PrefetchScalarGridSpec(
            num_scalar_prefetch=1,
            grid=(2, n2),
            in_specs=in_specs,
            out_specs=pl.BlockSpec((1, 1, _G, _C),
                                   lambda c, i, tr: (c, i, 0, 0)),
        ),
        out_shape=jax.ShapeDtypeStruct((2, n2, _G, _C), _F32),
        compiler_params=pltpu.CompilerParams(
            dimension_semantics=("parallel", "arbitrary")),
        name="embed_gather",
    )(tok, *([emb3] * _G))
    return out.reshape(_B, _T, _C)


def _pre_kernel(has_vmix, *refs):
    it = iter(refs)
    x_ref = next(it)
    vf_ref = next(it) if has_vmix else None
    mix_ref = next(it)
    vec_ref = next(it)
    w1_ref, w2_ref, a1_ref, a2_ref = next(it), next(it), next(it), next(it)
    if has_vmix:
        v1_ref, v2_ref = next(it), next(it)
    g1_ref, g2_ref = next(it), next(it)
    wr_ref, wk_ref, wv_ref, hm_ref = next(it), next(it), next(it), next(it)
    o_ref = next(it)
    prev_ref = next(it)

    j = pl.program_id(1)
    x = x_ref[0]
    xln = _ln(x, vec_ref[4], vec_ref[5])

    @pl.when(j == 0)
    def _():
        prev_ref[...] = jnp.zeros_like(prev_ref)

    xs = jnp.concatenate([prev_ref[...], xln[:_TP - 1]], axis=0)
    prev_ref[...] = xln[_TP - 1:_TP]
    xx = xs - xln
    mix = mix_ref[...]
    xr = xln + xx * mix[0]
    xw = xln + xx * mix[1]
    xk = xln + xx * mix[2]
    xv = xln + xx * mix[3]
    xa = xln + xx * mix[4]
    xg = xln + xx * mix[5]

    r = _dot(_bf(xr), wr_ref[...])
    k = _dot(_bf(xk), wk_ref[...])
    v = _dot(_bf(xv), wv_ref[...])

    wraw = vec_ref[0] + _dot(jnp.tanh(_dot(xw, w1_ref[...])), w2_ref[...])
    nwr = -wraw
    sp = jnp.maximum(nwr, 0.0) + jnp.log1p(jnp.exp(-jnp.abs(nwr)))
    ew = jnp.exp(-sp - 0.5)

    aa = jax.nn.sigmoid(vec_ref[1] + _dot(_dot(xa, a1_ref[...]), a2_ref[...]))
    g = _dot(jax.nn.sigmoid(_dot(xg, g1_ref[...])), g2_ref[...])
    if has_vmix:
        lam = jax.nn.sigmoid(vec_ref[6]
                             + _dot(_dot(xv, v1_ref[...]), v2_ref[...]))
        v = v + (vf_ref[0] - v) * lam

    kk = k * vec_ref[2]
    ss = _dot(kk * kk, hm_ref[...])
    kkn = kk / jnp.maximum(jnp.sqrt(ss), 1e-12)
    kf = k * (1.0 + (aa - 1.0) * vec_ref[3])

    o_ref[0, 0] = r
    o_ref[1, 0] = ew
    o_ref[2, 0] = kf
    o_ref[3, 0] = v
    o_ref[4, 0] = -kkn
    o_ref[5, 0] = kkn * aa
    o_ref[6, 0] = g


def _tmix_pre(x, tp, ln_g, ln_b, v_first):
    has_vmix = v_first is not None
    mixc = jnp.stack([tp['x_r'], tp['x_w'], tp['x_k'],
                      tp['x_v'], tp['x_a'], tp['x_g']])
    v0 = tp['v0'] if has_vmix else jnp.zeros((_C,), _F32)
    vecs = jnp.stack([tp['w0'], tp['a0'], tp['k_k'], tp['k_a'],
                      ln_g, ln_b, v0])

    act = pl.BlockSpec((1, _TP, _C), lambda bi, j: (bi, j, 0))
    full = lambda s: pl.BlockSpec(s, lambda bi, j: tuple([0] * len(s)))

    inputs = [x]
    in_specs = [act]
    if has_vmix:
        inputs.append(v_first)
        in_specs.append(act)
    inputs += [mixc, vecs, tp['w1'], tp['w2'], tp['a1'], tp['a2']]
    in_specs += [full((6, _C)), full((7, _C)), full((_C, 8)), full((8, _C)),
                 full((_C, 8)), full((8, _C))]
    if has_vmix:
        inputs += [tp['v1'], tp['v2']]
        in_specs += [full((_C, 8)), full((8, _C))]
    inputs += [tp['g1'], tp['g2'], _bf(tp['Wr']), _bf(tp['Wk']),
               _bf(tp['Wv']), _hmask()]
    in_specs += [full((_C, 8)), full((8, _C)), full((_C, _C)),
                 full((_C, _C)), full((_C, _C)), full((_C, _C))]

    outs = pl.pallas_call(
        functools.partial(_pre_kernel, has_vmix),
        grid=(_B, _T // _TP),
        in_specs=in_specs,
        out_specs=pl.BlockSpec((7, 1, _TP, _C), lambda bi, j: (0, bi, j, 0)),
        out_shape=jax.ShapeDtypeStruct((7, _B, _T, _C), _F32),
        scratch_shapes=[pltpu.VMEM((1, _C), _F32)],
        compiler_params=pltpu.CompilerParams(
            dimension_semantics=("parallel", "arbitrary")),
        name="tmix_pre",
    )(*inputs)
    return outs


def _scan_kernel(d_ref, m_ref, y_ref, s_ref):
    j = pl.program_id(1)

    @pl.when(j == 0)
    def _():
        s_ref[...] = jnp.zeros_like(s_ref)

    tri = jnp.broadcast_to(m_ref[0], (_H, _L, _L))
    strict = m_ref[1]
    incl = m_ref[0]
    eye = m_ref[2]

    for i in range(_TC // _L):
        sl = slice(i * _L, (i + 1) * _L)
        r = d_ref[0, 0, :, :, sl]
        ew = d_ref[1, 0, :, :, sl]
        k = d_ref[2, 0, :, :, sl]
        v = d_ref[3, 0, :, :, sl]
        a = d_ref[4, 0, :, :, sl]
        b = d_ref[5, 0, :, :, sl]

        sm = _bdot(ew, tri, 2, 2,
                   prec=jax.lax.Precision.HIGHEST)
        e1 = jnp.exp(sm)
        qn = 1.0 / e1
        ah = a * qn * jnp.exp(ew)
        bh = b * e1
        kh = k * e1
        rh = r * qn

        ab = _bdot(ah, bh, 1, 1) * strict
        akm = _bdot(ah, kh, 1, 1) * strict
        rbm = _bdot(rh, bh, 1, 1) * incl
        rkm = _bdot(rh, kh, 1, 1) * incl

        tm = eye + ab
        p = ab
        for _ in range(4):
            p = _bdot(p, p, 2, 1)
            tm = tm + _bdot(tm, p, 2, 1)

        st = s_ref[...]
        ut = _bdot(st, ah, 1, 1) + _bdot(v, akm, 2, 2)
        cct = _bdot(ut, tm, 2, 2)
        yt = (_bdot(st, rh, 1, 1) + _bdot(cct, rbm, 2, 2)
              + _bdot(v, rkm, 2, 2))
        y_ref[0, :, :, sl] = yt

        qlc = qn[:, :, _L - 1:_L]
        s_ref[...] = (st + _bdot(bh, cct, 2, 2)
                      + _bdot(kh, v, 2, 2)) * qlc


def _wkv7(pre):
    st = pre[:6]
    st = st.reshape(6, _B, _T, _H, _N).transpose(0, 1, 3, 4, 2)
    y = pl.pallas_call(
        _scan_kernel,
        grid=(_B, _T // _TC),
        in_specs=[
            pl.BlockSpec((6, 1, _H, _N, _TC),
                         lambda bi, j: (0, bi, 0, 0, j)),
            pl.BlockSpec((3, _L, _L), lambda bi, j: (0, 0, 0)),
        ],
        out_specs=pl.BlockSpec((1, _H, _N, _TC),
                               lambda bi, j: (bi, 0, 0, j)),
        out_shape=jax.ShapeDtypeStruct((_B, _H, _N, _T), _F32),
        scratch_shapes=[pltpu.VMEM((_H, _N, _N), _F32)],
        compiler_params=pltpu.CompilerParams(
            dimension_semantics=("parallel", "arbitrary")),
        name="wkv7_scan",
    )(st, _masks())
    return y.transpose(0, 3, 1, 2).reshape(_B, _T, _C)


def _post_kernel(y_ref, r_ref, k_ref, v_ref, g_ref, x_ref,
                 vec_ref, wo_ref, hm_ref, o_ref):
    y = y_ref[0]
    hm = hm_ref[...]
    m = _dot(y, hm) * (1.0 / _N)
    c = y - m
    var = _dot(c * c, hm) * (1.0 / _N)
    gn = c * jax.lax.rsqrt(var + 0.00064) * vec_ref[0] + vec_ref[1]
    rk = _dot(r_ref[0] * k_ref[0] * vec_ref[2], hm)
    y2 = gn + rk * v_ref[0]
    o_ref[0] = x_ref[0] + _dot(_bf(y2 * g_ref[0]), wo_ref[...])


def _tmix_post(y, r, k, v, g, x, tp):
    vecs = jnp.stack([tp['gn_g'], tp['gn_b'], tp['r_k'].reshape(_C)])
    act = pl.BlockSpec((1, _TP, _C), lambda bi, j: (bi, j, 0))
    full = lambda s: pl.BlockSpec(s, lambda bi, j: tuple([0] * len(s)))
    return pl.pallas_call(
        _post_kernel,
        grid=(_B, _T // _TP),
        in_specs=[act] * 6 + [full((3, _C)), full((_C, _C)), full((_C, _C))],
        out_specs=act,
        out_shape=jax.ShapeDtypeStruct((_B, _T, _C), _F32),
        compiler_params=pltpu.CompilerParams(
            dimension_semantics=("parallel", "arbitrary")),
        name="tmix_post",
    )(y, r, k, v, g, x, vecs, _bf(tp['Wo']), _hmask())


def _ffn_kernel(x_ref, vec_ref, wk_ref, wv_ref, o_ref, prev_ref):
    j = pl.program_id(1)
    x = x_ref[0]
    xln = _ln(x, vec_ref[0], vec_ref[1])

    @pl.when(j == 0)
    def _():
        prev_ref[...] = jnp.zeros_like(prev_ref)

    xs = jnp.concatenate([prev_ref[...], xln[:_TP - 1]], axis=0)
    prev_ref[...] = xln[_TP - 1:_TP]
    xk = xln + (xs - xln) * vec_ref[2]
    h = jnp.square(jnp.maximum(_dot(_bf(xk), wk_ref[...]), 0.0))
    o_ref[0] = x + _dot(_bf(h), wv_ref[...])


def _ffn(x, fp, ln_g, ln_b):
    vecs = jnp.stack([ln_g, ln_b, fp['x_k']])
    act = pl.BlockSpec((1, _TP, _C), lambda bi, j: (bi, j, 0))
    full = lambda s: pl.BlockSpec(s, lambda bi, j: tuple([0] * len(s)))
    return pl.pallas_call(
        _ffn_kernel,
        grid=(_B, _T // _TP),
        in_specs=[act, full((3, _C)), full((_C, 4 * _C)), full((4 * _C, _C))],
        out_specs=act,
        out_shape=jax.ShapeDtypeStruct((_B, _T, _C), _F32),
        scratch_shapes=[pltpu.VMEM((1, _C), _F32)],
        compiler_params=pltpu.CompilerParams(
            dimension_semantics=("parallel", "arbitrary")),
        name="ffn",
    )(x, vecs, _bf(fp['Wk']), _bf(fp['Wv']))


def _head_kernel(x_ref, lnw_ref, w_ref, b_ref, o_ref):
    xln = _ln(x_ref[...], lnw_ref[0], lnw_ref[1])
    o_ref[...] = _dot(_bf(xln), w_ref[...]) + b_ref[...]


def _head(x, ln_g, ln_b, wout, bout):
    x2 = x.reshape(_B * _T, _C)
    lnw = jnp.stack([ln_g, ln_b])
    out = pl.pallas_call(
        _head_kernel,
        grid=(_V // _VT, (_B * _T) // _TR),
        in_specs=[
            pl.BlockSpec((_TR, _C), lambda jv, i: (i, 0)),
            pl.BlockSpec((2, _C), lambda jv, i: (0, 0)),
            pl.BlockSpec((_C, _VT), lambda jv, i: (0, jv)),
            pl.BlockSpec((1, _VT), lambda jv, i: (0, jv)),
        ],
        out_specs=pl.BlockSpec((_TR, _VT), lambda jv, i: (i, jv)),
        out_shape=jax.ShapeDtypeStruct((_B * _T, _V), _F32),
        compiler_params=pltpu.CompilerParams(
            dimension_semantics=("parallel", "arbitrary")),
        name="head_proj",
    )(x2, lnw, _bf(wout), bout.reshape(1, _V))
    return out.reshape(_B, _T, _V)


def _layer(x, tp, fp, lna_g, lna_b, lnb_g, lnb_b, v_first):
    pre = _tmix_pre(x, tp, lna_g, lna_b, v_first)
    y = _wkv7(pre)
    x = _tmix_post(y, pre[0], pre[2], pre[3], pre[6], x, tp)
    x = _ffn(x, fp, lnb_g, lnb_b)
    return x, pre[3]


def kernel(tokens, params):
    p = params
    x = _embed(tokens, p['emb'])
    x, v_first = _layer(x, p['rwkv1'], p['ffn1'], p['ln1a_g'], p['ln1a_b'],
                        p['ln1b_g'], p['ln1b_b'], None)
    x, _ = _layer(x, p['rwkv2'], p['ffn2'], p['ln2a_g'], p['ln2a_b'],
                  p['ln2b_g'], p['ln2b_b'], v_first)
    return _head(x, p['lno_g'], p['lno_b'], p['Wout'], p['bout'])


# speedup vs baseline: 1.0726x; 1.0726x over previous
import functools

import numpy as np
import jax
import jax.numpy as jnp
from jax.experimental import pallas as pl
from jax.experimental.pallas import tpu as pltpu

_B, _T, _C, _V = 2, 2048, 512, 32000
_N = 16
_H = _C // _N
_L = 32
_TP = 256
_TC = 128
_TR = 256
_VT = 3200
_G = 8

_F32 = jnp.float32


def _hmask():
    return jnp.asarray(np.kron(np.eye(_H, dtype=np.float32),
                               np.ones((_N, _N), np.float32)))


def _masks():
    tri = np.tril(np.ones((_L, _L), np.float32))
    strict = np.tril(np.ones((_L, _L), np.float32), -1)
    eye = np.eye(_L, dtype=np.float32)
    return jnp.asarray(np.stack([tri, strict, eye]))


def _dot(a, b, prec=None):
    return jnp.dot(a, b, preferred_element_type=_F32, precision=prec)


def _bdot(a, b, ca, cb, prec=None):
    return jax.lax.dot_general(
        a, b, (((ca,), (cb,)), ((0,), (0,))),
        preferred_element_type=_F32, precision=prec)


_BF16 = jnp.bfloat16


def _bf(x):
    return x.astype(_BF16)


def _ln(x, g, b, eps=1e-5):
    m = jnp.mean(x, axis=-1, keepdims=True)
    c = x - m
    v = jnp.mean(c * c, axis=-1, keepdims=True)
    return c * jax.lax.rsqrt(v + eps) * g + b


def _embed_kernel(tok_ref, *refs):
    o_ref = refs[_G]
    for g in range(_G):
        o_ref[0, 0, g, :] = refs[g][0, 0, :]


def _embed(tokens, emb):
    tok = tokens.reshape(-1)
    n = tok.shape[0] // _G
    n2 = n // 2
    emb3 = emb.reshape(_V, 1, _C)

    def _imap(g, c, i, tr):
        return (tr[(c * n2 + i) * _G + g], 0, 0)

    in_specs = [pl.BlockSpec((1, 1, _C), functools.partial(_imap, g))
                for g in range(_G)]
    out = pl.pallas_call(
        _embed_kernel,
        grid_spec=pltpu.PrefetchScalarGridSpec(
            num_scalar_prefetch=1,
            grid=(2, n2),
            in_specs=in_specs,
            out_specs=pl.BlockSpec((1, 1, _G, _C),
                                   lambda c, i, tr: (c, i, 0, 0)),
        ),
        out_shape=jax.ShapeDtypeStruct((2, n2, _G, _C), _F32),
        compiler_params=pltpu.CompilerParams(
            dimension_semantics=("parallel", "arbitrary")),
        name="embed_gather",
    )(tok, *([emb3] * _G))
    return out.reshape(_B, _T, _C)


def _pre_kernel(has_vmix, *refs):
    it = iter(refs)
    x_ref = next(it)
    vf_ref = next(it) if has_vmix else None
    mix_ref = next(it)
    vec_ref = next(it)
    w1_ref, w2_ref, a1_ref, a2_ref = next(it), next(it), next(it), next(it)
    if has_vmix:
        v1_ref, v2_ref = next(it), next(it)
    g1_ref, g2_ref = next(it), next(it)
    wr_ref, wk_ref, wv_ref, hm_ref = next(it), next(it), next(it), next(it)
    o_ref = next(it)
    prev_ref = next(it)

    j = pl.program_id(1)
    x = x_ref[0]
    xln = _ln(x, vec_ref[4], vec_ref[5])

    @pl.when(j == 0)
    def _():
        prev_ref[...] = jnp.zeros_like(prev_ref)

    xs = jnp.concatenate([prev_ref[...], xln[:_TP - 1]], axis=0)
    prev_ref[...] = xln[_TP - 1:_TP]
    xx = xs - xln
    mix = mix_ref[...]
    xr = xln + xx * mix[0]
    xw = xln + xx * mix[1]
    xk = xln + xx * mix[2]
    xv = xln + xx * mix[3]
    xa = xln + xx * mix[4]
    xg = xln + xx * mix[5]

    r = _dot(_bf(xr), wr_ref[...])
    k = _dot(_bf(xk), wk_ref[...])
    v = _dot(_bf(xv), wv_ref[...])

    wraw = vec_ref[0] + _dot(jnp.tanh(_dot(xw, w1_ref[...])), w2_ref[...])
    nwr = -wraw
    sp = jnp.maximum(nwr, 0.0) + jnp.log1p(jnp.exp(-jnp.abs(nwr)))
    ew = jnp.exp(-sp - 0.5)

    aa = jax.nn.sigmoid(vec_ref[1] + _dot(_dot(xa, a1_ref[...]), a2_ref[...]))
    g = _dot(jax.nn.sigmoid(_dot(xg, g1_ref[...])), g2_ref[...])
    if has_vmix:
        lam = jax.nn.sigmoid(vec_ref[6]
                             + _dot(_dot(xv, v1_ref[...]), v2_ref[...]))
        v = v + (vf_ref[0] - v) * lam

    kk = k * vec_ref[2]
    ss = _dot(kk * kk, hm_ref[...])
    kkn = kk / jnp.maximum(jnp.sqrt(ss), 1e-12)
    kf = k * (1.0 + (aa - 1.0) * vec_ref[3])

    o_ref[0, 0] = r
    o_ref[1, 0] = ew
    o_ref[2, 0] = kf
    o_ref[3, 0] = v
    o_ref[4, 0] = -kkn
    o_ref[5, 0] = kkn * aa
    o_ref[6, 0] = g


def _tmix_pre(x, tp, ln_g, ln_b, v_first):
    has_vmix = v_first is not None
    mixc = jnp.stack([tp['x_r'], tp['x_w'], tp['x_k'],
                      tp['x_v'], tp['x_a'], tp['x_g']])
    v0 = tp['v0'] if has_vmix else jnp.zeros((_C,), _F32)
    vecs = jnp.stack([tp['w0'], tp['a0'], tp['k_k'], tp['k_a'],
                      ln_g, ln_b, v0])

    act = pl.BlockSpec((1, _TP, _C), lambda bi, j: (bi, j, 0))
    full = lambda s: pl.BlockSpec(s, lambda bi, j: tuple([0] * len(s)))

    inputs = [x]
    in_specs = [act]
    if has_vmix:
        inputs.append(v_first)
        in_specs.append(act)
    inputs += [mixc, vecs, tp['w1'], tp['w2'], tp['a1'], tp['a2']]
    in_specs += [full((6, _C)), full((7, _C)), full((_C, 8)), full((8, _C)),
                 full((_C, 8)), full((8, _C))]
    if has_vmix:
        inputs += [tp['v1'], tp['v2']]
        in_specs += [full((_C, 8)), full((8, _C))]
    inputs += [tp['g1'], tp['g2'], _bf(tp['Wr']), _bf(tp['Wk']),
               _bf(tp['Wv']), _hmask()]
    in_specs += [full((_C, 8)), full((8, _C)), full((_C, _C)),
                 full((_C, _C)), full((_C, _C)), full((_C, _C))]

    outs = pl.pallas_call(
        functools.partial(_pre_kernel, has_vmix),
        grid=(_B, _T // _TP),
        in_specs=in_specs,
        out_specs=pl.BlockSpec((7, 1, _TP, _C), lambda bi, j: (0, bi, j, 0)),
        out_shape=jax.ShapeDtypeStruct((7, _B, _T, _C), _F32),
        scratch_shapes=[pltpu.VMEM((1, _C), _F32)],
        compiler_params=pltpu.CompilerParams(
            dimension_semantics=("parallel", "arbitrary")),
        name="tmix_pre",
    )(*inputs)
    return outs


def _scan_kernel(d_ref, m_ref, y_ref, s_ref):
    j = pl.program_id(1)

    @pl.when(j == 0)
    def _():
        s_ref[...] = jnp.zeros_like(s_ref)

    tri = jnp.broadcast_to(m_ref[0], (_H, _L, _L))
    strict = m_ref[1]
    incl = m_ref[0]
    eye = m_ref[2]

    for i in range(_TC // _L):
        sl = slice(i * _L, (i + 1) * _L)
        r = d_ref[0, 0, :, :, sl]
        ew = d_ref[1, 0, :, :, sl]
        k = d_ref[2, 0, :, :, sl]
        v = d_ref[3, 0, :, :, sl]
        a = d_ref[4, 0, :, :, sl]
        b = d_ref[5, 0, :, :, sl]

        sm = _bdot(ew, tri, 2, 2)
        e1 = jnp.exp(sm)
        qn = 1.0 / e1
        ah = a * qn * jnp.exp(ew)
        bh = b * e1
        kh = k * e1
        rh = r * qn

        ab = _bdot(ah, bh, 1, 1) * strict
        akm = _bdot(ah, kh, 1, 1) * strict
        rbm = _bdot(rh, bh, 1, 1) * incl
        rkm = _bdot(rh, kh, 1, 1) * incl

        tm = eye + ab
        p = ab
        for _ in range(4):
            p = _bdot(p, p, 2, 1)
            tm = tm + _bdot(tm, p, 2, 1)

        st = s_ref[...]
        ut = _bdot(st, ah, 1, 1) + _bdot(v, akm, 2, 2)
        cct = _bdot(ut, tm, 2, 2)
        yt = (_bdot(st, rh, 1, 1) + _bdot(cct, rbm, 2, 2)
              + _bdot(v, rkm, 2, 2))
        y_ref[0, :, :, sl] = yt

        qlc = qn[:, :, _L - 1:_L]
        s_ref[...] = (st + _bdot(bh, cct, 2, 2)
                      + _bdot(kh, v, 2, 2)) * qlc


def _wkv7(pre):
    st = pre[:6]
    st = st.reshape(6, _B, _T, _H, _N).transpose(0, 1, 3, 4, 2)
    y = pl.pallas_call(
        _scan_kernel,
        grid=(_B, _T // _TC),
        in_specs=[
            pl.BlockSpec((6, 1, _H, _N, _TC),
                         lambda bi, j: (0, bi, 0, 0, j)),
            pl.BlockSpec((3, _L, _L), lambda bi, j: (0, 0, 0)),
        ],
        out_specs=pl.BlockSpec((1, _H, _N, _TC),
                               lambda bi, j: (bi, 0, 0, j)),
        out_shape=jax.ShapeDtypeStruct((_B, _H, _N, _T), _F32),
        scratch_shapes=[pltpu.VMEM((_H, _N, _N), _F32)],
        compiler_params=pltpu.CompilerParams(
            dimension_semantics=("parallel", "arbitrary")),
        name="wkv7_scan",
    )(st, _masks())
    return y.transpose(0, 3, 1, 2).reshape(_B, _T, _C)


def _post_kernel(y_ref, r_ref, k_ref, v_ref, g_ref, x_ref,
                 vec_ref, wo_ref, hm_ref, o_ref):
    y = y_ref[0]
    hm = hm_ref[...]
    m = _dot(y, hm) * (1.0 / _N)
    c = y - m
    var = _dot(c * c, hm) * (1.0 / _N)
    gn = c * jax.lax.rsqrt(var + 0.00064) * vec_ref[0] + vec_ref[1]
    rk = _dot(r_ref[0] * k_ref[0] * vec_ref[2], hm)
    y2 = gn + rk * v_ref[0]
    o_ref[0] = x_ref[0] + _dot(_bf(y2 * g_ref[0]), wo_ref[...])


def _tmix_post(y, r, k, v, g, x, tp):
    vecs = jnp.stack([tp['gn_g'], tp['gn_b'], tp['r_k'].reshape(_C)])
    act = pl.BlockSpec((1, _TP, _C), lambda bi, j: (bi, j, 0))
    full = lambda s: pl.BlockSpec(s, lambda bi, j: tuple([0] * len(s)))
    return pl.pallas_call(
        _post_kernel,
        grid=(_B, _T // _TP),
        in_specs=[act] * 6 + [full((3, _C)), full((_C, _C)), full((_C, _C))],
        out_specs=act,
        out_shape=jax.ShapeDtypeStruct((_B, _T, _C), _F32),
        compiler_params=pltpu.CompilerParams(
            dimension_semantics=("parallel", "arbitrary")),
        name="tmix_post",
    )(y, r, k, v, g, x, vecs, _bf(tp['Wo']), _hmask())


def _ffn_kernel(x_ref, vec_ref, wk_ref, wv_ref, o_ref, prev_ref):
    j = pl.program_id(1)
    x = x_ref[0]
    xln = _ln(x, vec_ref[0], vec_ref[1])

    @pl.when(j == 0)
    def _():
        prev_ref[...] = jnp.zeros_like(prev_ref)

    xs = jnp.concatenate([prev_ref[...], xln[:_TP - 1]], axis=0)
    prev_ref[...] = xln[_TP - 1:_TP]
    xk = xln + (xs - xln) * vec_ref[2]
    h = jnp.square(jnp.maximum(_dot(_bf(xk), wk_ref[...]), 0.0))
    o_ref[0] = x + _dot(_bf(h), wv_ref[...])


def _ffn(x, fp, ln_g, ln_b):
    vecs = jnp.stack([ln_g, ln_b, fp['x_k']])
    act = pl.BlockSpec((1, _TP, _C), lambda bi, j: (bi, j, 0))
    full = lambda s: pl.BlockSpec(s, lambda bi, j: tuple([0] * len(s)))
    return pl.pallas_call(
        _ffn_kernel,
        grid=(_B, _T // _TP),
        in_specs=[act, full((3, _C)), full((_C, 4 * _C)), full((4 * _C, _C))],
        out_specs=act,
        out_shape=jax.ShapeDtypeStruct((_B, _T, _C), _F32),
        scratch_shapes=[pltpu.VMEM((1, _C), _F32)],
        compiler_params=pltpu.CompilerParams(
            dimension_semantics=("parallel", "arbitrary")),
        name="ffn",
    )(x, vecs, _bf(fp['Wk']), _bf(fp['Wv']))


def _head_kernel(x_ref, lnw_ref, w_ref, b_ref, o_ref):
    xln = _ln(x_ref[...], lnw_ref[0], lnw_ref[1])
    o_ref[...] = _dot(_bf(xln), w_ref[...]) + b_ref[...]


def _head(x, ln_g, ln_b, wout, bout):
    x2 = x.reshape(_B * _T, _C)
    lnw = jnp.stack([ln_g, ln_b])
    out = pl.pallas_call(
        _head_kernel,
        grid=(_V // _VT, (_B * _T) // _TR),
        in_specs=[
            pl.BlockSpec((_TR, _C), lambda jv, i: (i, 0)),
            pl.BlockSpec((2, _C), lambda jv, i: (0, 0)),
            pl.BlockSpec((_C, _VT), lambda jv, i: (0, jv)),
            pl.BlockSpec((1, _VT), lambda jv, i: (0, jv)),
        ],
        out_specs=pl.BlockSpec((_TR, _VT), lambda jv, i: (i, jv)),
        out_shape=jax.ShapeDtypeStruct((_B * _T, _V), _F32),
        compiler_params=pltpu.CompilerParams(
            dimension_semantics=("parallel", "arbitrary")),
        name="head_proj",
    )(x2, lnw, _bf(wout), bout.reshape(1, _V))
    return out.reshape(_B, _T, _V)


def _layer(x, tp, fp, lna_g, lna_b, lnb_g, lnb_b, v_first):
    pre = _tmix_pre(x, tp, lna_g, lna_b, v_first)
    y = _wkv7(pre)
    x = _tmix_post(y, pre[0], pre[2], pre[3], pre[6], x, tp)
    x = _ffn(x, fp, lnb_g, lnb_b)
    return x, pre[3]


def kernel(tokens, params):
    p = params
    x = _embed(tokens, p['emb'])
    x, v_first = _layer(x, p['rwkv1'], p['ffn1'], p['ln1a_g'], p['ln1a_b'],
                        p['ln1b_g'], p['ln1b_b'], None)
    x, _ = _layer(x, p['rwkv2'], p['ffn2'], p['ln2a_g'], p['ln2a_b'],
                  p['ln2b_g'], p['ln2b_b'], v_first)
    return _head(x, p['lno_g'], p['lno_b'], p['Wout'], p['bout'])


# speedup vs baseline: 1.2643x; 1.1787x over previous
import functools

import numpy as np
import jax
import jax.numpy as jnp
from jax.experimental import pallas as pl
from jax.experimental.pallas import tpu as pltpu

_B, _T, _C, _V = 2, 2048, 512, 32000
_N = 16
_H = _C // _N
_L = 32
_TP = 256
_TC = 128
_TR = 256
_VT = 3200
_G = 16

_F32 = jnp.float32


def _hmask():
    return jnp.asarray(np.kron(np.eye(_H, dtype=np.float32),
                               np.ones((_N, _N), np.float32)))


def _masks():
    tri = np.tril(np.ones((_L, _L), np.float32))
    strict = np.tril(np.ones((_L, _L), np.float32), -1)
    eye = np.eye(_L, dtype=np.float32)
    return jnp.asarray(np.stack([tri, strict, eye]))


def _dot(a, b, prec=None):
    return jnp.dot(a, b, preferred_element_type=_F32, precision=prec)


def _bdot(a, b, ca, cb, prec=None):
    return jax.lax.dot_general(
        a, b, (((ca,), (cb,)), ((0,), (0,))),
        preferred_element_type=_F32, precision=prec)


_BF16 = jnp.bfloat16


def _bf(x):
    return x.astype(_BF16)


def _ln(x, g, b, eps=1e-5):
    m = jnp.mean(x, axis=-1, keepdims=True)
    c = x - m
    v = jnp.mean(c * c, axis=-1, keepdims=True)
    return c * jax.lax.rsqrt(v + eps) * g + b


def _embed_kernel(tok_ref, *refs):
    o_ref = refs[_G]
    for g in range(_G):
        o_ref[0, 0, g, :] = refs[g][0, 0, :]


def _embed(tokens, emb):
    tok = tokens.reshape(-1)
    n = tok.shape[0] // _G
    n2 = n // 2
    emb3 = emb.reshape(_V, 1, _C)

    def _imap(g, c, i, tr):
        return (tr[(c * n2 + i) * _G + g], 0, 0)

    in_specs = [pl.BlockSpec((1, 1, _C), functools.partial(_imap, g))
                for g in range(_G)]
    out = pl.pallas_call(
        _embed_kernel,
        grid_spec=pltpu.PrefetchScalarGridSpec(
            num_scalar_prefetch=1,
            grid=(2, n2),
            in_specs=in_specs,
            out_specs=pl.BlockSpec((1, 1, _G, _C),
                                   lambda c, i, tr: (c, i, 0, 0)),
        ),
        out_shape=jax.ShapeDtypeStruct((2, n2, _G, _C), _F32),
        compiler_params=pltpu.CompilerParams(
            dimension_semantics=("parallel", "arbitrary")),
        name="embed_gather",
    )(tok, *([emb3] * _G))
    return out.reshape(_B, _T, _C)


def _pre_kernel(has_vmix, *refs):
    it = iter(refs)
    x_ref = next(it)
    vf_ref = next(it) if has_vmix else None
    mix_ref = next(it)
    vec_ref = next(it)
    w1_ref, w2_ref, a1_ref, a2_ref = next(it), next(it), next(it), next(it)
    if has_vmix:
        v1_ref, v2_ref = next(it), next(it)
    g1_ref, g2_ref = next(it), next(it)
    wr_ref, wk_ref, wv_ref, hm_ref = next(it), next(it), next(it), next(it)
    o_ref = next(it)
    prev_ref = next(it)

    j = pl.program_id(1)
    x = x_ref[0]
    xln = _ln(x, vec_ref[4], vec_ref[5])

    @pl.when(j == 0)
    def _():
        prev_ref[...] = jnp.zeros_like(prev_ref)

    xs = jnp.concatenate([prev_ref[...], xln[:_TP - 1]], axis=0)
    prev_ref[...] = xln[_TP - 1:_TP]
    xx = xs - xln
    mix = mix_ref[...]
    xr = xln + xx * mix[0]
    xw = xln + xx * mix[1]
    xk = xln + xx * mix[2]
    xv = xln + xx * mix[3]
    xa = xln + xx * mix[4]
    xg = xln + xx * mix[5]

    r = _dot(_bf(xr), wr_ref[...])
    k = _dot(_bf(xk), wk_ref[...])
    v = _dot(_bf(xv), wv_ref[...])

    wraw = vec_ref[0] + _dot(jnp.tanh(_dot(xw, w1_ref[...])), w2_ref[...])
    nwr = -wraw
    sp = jnp.maximum(nwr, 0.0) + jnp.log1p(jnp.exp(-jnp.abs(nwr)))
    ew = jnp.exp(-sp - 0.5)

    aa = jax.nn.sigmoid(vec_ref[1] + _dot(_dot(xa, a1_ref[...]), a2_ref[...]))
    g = _dot(jax.nn.sigmoid(_dot(xg, g1_ref[...])), g2_ref[...])
    if has_vmix:
        vf = vf_ref[0].reshape(_C, _TP).T
        lam = jax.nn.sigmoid(vec_ref[6]
                             + _dot(_dot(xv, v1_ref[...]), v2_ref[...]))
        v = v + (vf - v) * lam

    kk = k * vec_ref[2]
    ss = _dot(kk * kk, hm_ref[...])
    kkn = kk / jnp.maximum(jnp.sqrt(ss), 1e-12)
    kf = k * (1.0 + (aa - 1.0) * vec_ref[3])

    for q, val in enumerate((r, ew, kf, v, -kkn, kkn * aa, g)):
        o_ref[q, 0] = val.T.reshape(_H, _N, _TP)


def _tmix_pre(x, tp, ln_g, ln_b, v_first):
    has_vmix = v_first is not None
    mixc = jnp.stack([tp['x_r'], tp['x_w'], tp['x_k'],
                      tp['x_v'], tp['x_a'], tp['x_g']])
    v0 = tp['v0'] if has_vmix else jnp.zeros((_C,), _F32)
    vecs = jnp.stack([tp['w0'], tp['a0'], tp['k_k'], tp['k_a'],
                      ln_g, ln_b, v0])

    act = pl.BlockSpec((1, _TP, _C), lambda bi, j: (bi, j, 0))
    actT = pl.BlockSpec((1, _H, _N, _TP), lambda bi, j: (bi, 0, 0, j))
    full = lambda s: pl.BlockSpec(s, lambda bi, j: tuple([0] * len(s)))

    inputs = [x]
    in_specs = [act]
    if has_vmix:
        inputs.append(v_first)
        in_specs.append(actT)
    inputs += [mixc, vecs, tp['w1'], tp['w2'], tp['a1'], tp['a2']]
    in_specs += [full((6, _C)), full((7, _C)), full((_C, 8)), full((8, _C)),
                 full((_C, 8)), full((8, _C))]
    if has_vmix:
        inputs += [tp['v1'], tp['v2']]
        in_specs += [full((_C, 8)), full((8, _C))]
    inputs += [tp['g1'], tp['g2'], _bf(tp['Wr']), _bf(tp['Wk']),
               _bf(tp['Wv']), _hmask()]
    in_specs += [full((_C, 8)), full((8, _C)), full((_C, _C)),
                 full((_C, _C)), full((_C, _C)), full((_C, _C))]

    outs = pl.pallas_call(
        functools.partial(_pre_kernel, has_vmix),
        grid=(_B, _T // _TP),
        in_specs=in_specs,
        out_specs=pl.BlockSpec((7, 1, _H, _N, _TP),
                               lambda bi, j: (0, bi, 0, 0, j)),
        out_shape=jax.ShapeDtypeStruct((7, _B, _H, _N, _T), _F32),
        scratch_shapes=[pltpu.VMEM((1, _C), _F32)],
        compiler_params=pltpu.CompilerParams(
            dimension_semantics=("parallel", "arbitrary")),
        name="tmix_pre",
    )(*inputs)
    return outs


def _scan_kernel(d_ref, m_ref, y_ref, s_ref):
    j = pl.program_id(1)

    @pl.when(j == 0)
    def _():
        s_ref[...] = jnp.zeros_like(s_ref)

    tri = jnp.broadcast_to(m_ref[0], (_H, _L, _L))
    strict = m_ref[1]
    incl = m_ref[0]
    eye = m_ref[2]

    for i in range(_TC // _L):
        sl = slice(i * _L, (i + 1) * _L)
        r = d_ref[0, 0, :, :, sl]
        ew = d_ref[1, 0, :, :, sl]
        k = d_ref[2, 0, :, :, sl]
        v = d_ref[3, 0, :, :, sl]
        a = d_ref[4, 0, :, :, sl]
        b = d_ref[5, 0, :, :, sl]

        sm = _bdot(ew, tri, 2, 2)
        e1 = jnp.exp(sm)
        qn = 1.0 / e1
        ah = a * qn * jnp.exp(ew)
        bh = b * e1
        kh = k * e1
        rh = r * qn

        ab = _bdot(ah, bh, 1, 1) * strict
        akm = _bdot(ah, kh, 1, 1) * strict
        rbm = _bdot(rh, bh, 1, 1) * incl
        rkm = _bdot(rh, kh, 1, 1) * incl

        tm = eye + ab
        p = ab
        for _ in range(4):
            p = _bdot(p, p, 2, 1)
            tm = tm + _bdot(tm, p, 2, 1)

        st = s_ref[...]
        ut = _bdot(st, ah, 1, 1) + _bdot(v, akm, 2, 2)
        cct = _bdot(ut, tm, 2, 2)
        yt = (_bdot(st, rh, 1, 1) + _bdot(cct, rbm, 2, 2)
              + _bdot(v, rkm, 2, 2))
        y_ref[0, :, :, sl] = yt

        qlc = qn[:, :, _L - 1:_L]
        s_ref[...] = (st + _bdot(bh, cct, 2, 2)
                      + _bdot(kh, v, 2, 2)) * qlc


def _wkv7(pre):
    return pl.pallas_call(
        _scan_kernel,
        grid=(_B, _T // _TC),
        in_specs=[
            pl.BlockSpec((7, 1, _H, _N, _TC),
                         lambda bi, j: (0, bi, 0, 0, j)),
            pl.BlockSpec((3, _L, _L), lambda bi, j: (0, 0, 0)),
        ],
        out_specs=pl.BlockSpec((1, _H, _N, _TC),
                               lambda bi, j: (bi, 0, 0, j)),
        out_shape=jax.ShapeDtypeStruct((_B, _H, _N, _T), _F32),
        scratch_shapes=[pltpu.VMEM((_H, _N, _N), _F32)],
        compiler_params=pltpu.CompilerParams(
            dimension_semantics=("parallel", "arbitrary")),
        name="wkv7_scan",
    )(pre, _masks())


def _post_kernel(y_ref, p_ref, x_ref, vec_ref, wot_ref, hm_ref, o_ref):
    y = y_ref[0].reshape(_C, _TP)
    r = p_ref[0, 0].reshape(_C, _TP)
    k = p_ref[2, 0].reshape(_C, _TP)
    v = p_ref[3, 0].reshape(_C, _TP)
    g = p_ref[6, 0].reshape(_C, _TP)
    hm = hm_ref[...]
    m = _dot(hm, _bf(y)) * (1.0 / _N)
    c = y - m
    var = _dot(hm, _bf(c * c)) * (1.0 / _N)
    gn = (c * jax.lax.rsqrt(var + 0.00064) * vec_ref[:, 0:1]
          + vec_ref[:, 1:2])
    rk = _dot(hm, _bf(r * k * vec_ref[:, 2:3]))
    y2 = gn + rk * v
    o_ct = _dot(wot_ref[...], _bf(y2 * g))
    o_ref[0] = x_ref[0] + o_ct.T


def _tmix_post(y, pre, x, tp):
    vecs = jnp.stack([tp['gn_g'], tp['gn_b'], tp['r_k'].reshape(_C)], axis=1)
    act = pl.BlockSpec((1, _TP, _C), lambda bi, j: (bi, j, 0))
    actT = pl.BlockSpec((1, _H, _N, _TP), lambda bi, j: (bi, 0, 0, j))
    preT = pl.BlockSpec((7, 1, _H, _N, _TP), lambda bi, j: (0, bi, 0, 0, j))
    full = lambda s: pl.BlockSpec(s, lambda bi, j: tuple([0] * len(s)))
    return pl.pallas_call(
        _post_kernel,
        grid=(_B, _T // _TP),
        in_specs=[actT, preT, act, full((_C, 3)), full((_C, _C)),
                  full((_C, _C))],
        out_specs=act,
        out_shape=jax.ShapeDtypeStruct((_B, _T, _C), _F32),
        compiler_params=pltpu.CompilerParams(
            dimension_semantics=("parallel", "arbitrary")),
        name="tmix_post",
    )(y, pre, x, vecs, _bf(tp['Wo'].T), _bf(_hmask()))


def _ffn_kernel(x_ref, vec_ref, wk_ref, wv_ref, o_ref, prev_ref):
    j = pl.program_id(1)
    x = x_ref[0]
    xln = _ln(x, vec_ref[0], vec_ref[1])

    @pl.when(j == 0)
    def _():
        prev_ref[...] = jnp.zeros_like(prev_ref)

    xs = jnp.concatenate([prev_ref[...], xln[:_TP - 1]], axis=0)
    prev_ref[...] = xln[_TP - 1:_TP]
    xk = xln + (xs - xln) * vec_ref[2]
    h = jnp.square(jnp.maximum(_dot(_bf(xk), wk_ref[...]), 0.0))
    o_ref[0] = x + _dot(_bf(h), wv_ref[...])


def _ffn(x, fp, ln_g, ln_b):
    vecs = jnp.stack([ln_g, ln_b, fp['x_k']])
    act = pl.BlockSpec((1, _TP, _C), lambda bi, j: (bi, j, 0))
    full = lambda s: pl.BlockSpec(s, lambda bi, j: tuple([0] * len(s)))
    return pl.pallas_call(
        _ffn_kernel,
        grid=(_B, _T // _TP),
        in_specs=[act, full((3, _C)), full((_C, 4 * _C)), full((4 * _C, _C))],
        out_specs=act,
        out_shape=jax.ShapeDtypeStruct((_B, _T, _C), _F32),
        scratch_shapes=[pltpu.VMEM((1, _C), _F32)],
        compiler_params=pltpu.CompilerParams(
            dimension_semantics=("parallel", "arbitrary")),
        name="ffn",
    )(x, vecs, _bf(fp['Wk']), _bf(fp['Wv']))


def _head_kernel(x_ref, lnw_ref, w_ref, b_ref, o_ref):
    xln = _ln(x_ref[...], lnw_ref[0], lnw_ref[1])
    o_ref[...] = _dot(_bf(xln), w_ref[...]) + b_ref[...]


def _head(x, ln_g, ln_b, wout, bout):
    x2 = x.reshape(_B * _T, _C)
    lnw = jnp.stack([ln_g, ln_b])
    out = pl.pallas_call(
        _head_kernel,
        grid=(_V // _VT, (_B * _T) // _TR),
        in_specs=[
            pl.BlockSpec((_TR, _C), lambda jv, i: (i, 0)),
            pl.BlockSpec((2, _C), lambda jv, i: (0, 0)),
            pl.BlockSpec((_C, _VT), lambda jv, i: (0, jv)),
            pl.BlockSpec((1, _VT), lambda jv, i: (0, jv)),
        ],
        out_specs=pl.BlockSpec((_TR, _VT), lambda jv, i: (i, jv)),
        out_shape=jax.ShapeDtypeStruct((_B * _T, _V), _F32),
        compiler_params=pltpu.CompilerParams(
            dimension_semantics=("parallel", "arbitrary")),
        name="head_proj",
    )(x2, lnw, _bf(wout), bout.reshape(1, _V))
    return out.reshape(_B, _T, _V)


def _layer(x, tp, fp, lna_g, lna_b, lnb_g, lnb_b, v_first):
    pre = _tmix_pre(x, tp, lna_g, lna_b, v_first)
    y = _wkv7(pre)
    x = _tmix_post(y, pre, x, tp)
    x = _ffn(x, fp, lnb_g, lnb_b)
    return x, pre[3]


def kernel(tokens, params):
    p = params
    x = _embed(tokens, p['emb'])
    x, v_first = _layer(x, p['rwkv1'], p['ffn1'], p['ln1a_g'], p['ln1a_b'],
                        p['ln1b_g'], p['ln1b_b'], None)
    x, _ = _layer(x, p['rwkv2'], p['ffn2'], p['ln2a_g'], p['ln2a_b'],
                  p['ln2b_g'], p['ln2b_b'], v_first)
    return _head(x, p['lno_g'], p['lno_b'], p['Wout'], p['bout'])


# speedup vs baseline: 1.3644x; 1.0792x over previous
import functools

import numpy as np
import jax
import jax.numpy as jnp
from jax.experimental import pallas as pl
from jax.experimental.pallas import tpu as pltpu

_B, _T, _C, _V = 2, 2048, 512, 32000
_N = 16
_H = _C // _N
_L = 32
_TP = 256
_TC = 128
_TR = 256
_VT = 3200
_G = 16

_F32 = jnp.float32


def _hmask():
    return jnp.asarray(np.kron(np.eye(_H, dtype=np.float32),
                               np.ones((_N, _N), np.float32)))


def _masks():
    tri = np.tril(np.ones((_L, _L), np.float32))
    strict = np.tril(np.ones((_L, _L), np.float32), -1)
    eye = np.eye(_L, dtype=np.float32)
    return jnp.asarray(np.stack([tri, strict, eye]))


def _dot(a, b, prec=None):
    return jnp.dot(a, b, preferred_element_type=_F32, precision=prec)


def _bdot(a, b, ca, cb, prec=None):
    return jax.lax.dot_general(
        a, b, (((ca,), (cb,)), ((0,), (0,))),
        preferred_element_type=_F32, precision=prec)


_BF16 = jnp.bfloat16


def _bf(x):
    return x.astype(_BF16)


def _ln(x, g, b, eps=1e-5):
    m = jnp.mean(x, axis=-1, keepdims=True)
    c = x - m
    v = jnp.mean(c * c, axis=-1, keepdims=True)
    return c * jax.lax.rsqrt(v + eps) * g + b


def _embed_kernel(tok_ref, *refs):
    o_ref = refs[_G]
    for g in range(_G):
        o_ref[0, 0, g, :] = refs[g][0, 0, :]


def _embed(tokens, emb):
    tok = tokens.reshape(-1)
    n = tok.shape[0] // _G
    n2 = n // 2
    emb3 = emb.reshape(_V, 1, _C)

    def _imap(g, c, i, tr):
        return (tr[(c * n2 + i) * _G + g], 0, 0)

    in_specs = [pl.BlockSpec((1, 1, _C), functools.partial(_imap, g))
                for g in range(_G)]
    out = pl.pallas_call(
        _embed_kernel,
        grid_spec=pltpu.PrefetchScalarGridSpec(
            num_scalar_prefetch=1,
            grid=(2, n2),
            in_specs=in_specs,
            out_specs=pl.BlockSpec((1, 1, _G, _C),
                                   lambda c, i, tr: (c, i, 0, 0)),
        ),
        out_shape=jax.ShapeDtypeStruct((2, n2, _G, _C), _F32),
        compiler_params=pltpu.CompilerParams(
            dimension_semantics=("parallel", "arbitrary")),
        name="embed_gather",
    )(tok, *([emb3] * _G))
    return out.reshape(_B, _T, _C)


def _pre_kernel(has_vmix, *refs):
    it = iter(refs)
    x_ref = next(it)
    vf_ref = next(it) if has_vmix else None
    mix_ref = next(it)
    vec_ref = next(it)
    w1_ref, w2_ref, a1_ref, a2_ref = next(it), next(it), next(it), next(it)
    if has_vmix:
        v1_ref, v2_ref = next(it), next(it)
    g1_ref, g2_ref = next(it), next(it)
    wr_ref, wk_ref, wv_ref, hm_ref = next(it), next(it), next(it), next(it)
    o_ref = next(it)
    prev_ref = next(it)

    j = pl.program_id(1)
    x = x_ref[0]
    xln = _ln(x, vec_ref[4], vec_ref[5])

    @pl.when(j == 0)
    def _():
        prev_ref[...] = jnp.zeros_like(prev_ref)

    xs = jnp.concatenate([prev_ref[...], xln[:_TP - 1]], axis=0)
    prev_ref[...] = xln[_TP - 1:_TP]
    xx = xs - xln
    mix = mix_ref[...]
    xr = xln + xx * mix[0]
    xw = xln + xx * mix[1]
    xk = xln + xx * mix[2]
    xv = xln + xx * mix[3]
    xa = xln + xx * mix[4]
    xg = xln + xx * mix[5]

    r = _dot(_bf(xr), wr_ref[...])
    k = _dot(_bf(xk), wk_ref[...])
    v = _dot(_bf(xv), wv_ref[...])

    wraw = vec_ref[0] + _dot(jnp.tanh(_dot(xw, w1_ref[...])), w2_ref[...])
    nwr = -wraw
    sp = jnp.maximum(nwr, 0.0) + jnp.log1p(jnp.exp(-jnp.abs(nwr)))
    ew = jnp.exp(-sp - 0.5)

    aa = jax.nn.sigmoid(vec_ref[1] + _dot(_dot(xa, a1_ref[...]), a2_ref[...]))
    g = _dot(jax.nn.sigmoid(_dot(xg, g1_ref[...])), g2_ref[...])
    if has_vmix:
        vf = vf_ref[0].reshape(_C, _TP).T
        lam = jax.nn.sigmoid(vec_ref[6]
                             + _dot(_dot(xv, v1_ref[...]), v2_ref[...]))
        v = v + (vf - v) * lam

    kk = k * vec_ref[2]
    ss = _dot(kk * kk, hm_ref[...])
    kkn = kk / jnp.maximum(jnp.sqrt(ss), 1e-12)
    kf = k * (1.0 + (aa - 1.0) * vec_ref[3])

    for q, val in enumerate((r, ew, kf, v, -kkn, kkn * aa, g)):
        o_ref[q, 0] = val.T.reshape(_H, _N, _TP)


def _tmix_pre(x, tp, ln_g, ln_b, v_first):
    has_vmix = v_first is not None
    mixc = jnp.stack([tp['x_r'], tp['x_w'], tp['x_k'],
                      tp['x_v'], tp['x_a'], tp['x_g']])
    v0 = tp['v0'] if has_vmix else jnp.zeros((_C,), _F32)
    vecs = jnp.stack([tp['w0'], tp['a0'], tp['k_k'], tp['k_a'],
                      ln_g, ln_b, v0])

    act = pl.BlockSpec((1, _TP, _C), lambda bi, j: (bi, j, 0))
    actT = pl.BlockSpec((1, _H, _N, _TP), lambda bi, j: (bi, 0, 0, j))
    full = lambda s: pl.BlockSpec(s, lambda bi, j: tuple([0] * len(s)))

    inputs = [x]
    in_specs = [act]
    if has_vmix:
        inputs.append(v_first)
        in_specs.append(actT)
    inputs += [mixc, vecs, tp['w1'], tp['w2'], tp['a1'], tp['a2']]
    in_specs += [full((6, _C)), full((7, _C)), full((_C, 8)), full((8, _C)),
                 full((_C, 8)), full((8, _C))]
    if has_vmix:
        inputs += [tp['v1'], tp['v2']]
        in_specs += [full((_C, 8)), full((8, _C))]
    inputs += [tp['g1'], tp['g2'], _bf(tp['Wr']), _bf(tp['Wk']),
               _bf(tp['Wv']), _hmask()]
    in_specs += [full((_C, 8)), full((8, _C)), full((_C, _C)),
                 full((_C, _C)), full((_C, _C)), full((_C, _C))]

    outs = pl.pallas_call(
        functools.partial(_pre_kernel, has_vmix),
        grid=(_B, _T // _TP),
        in_specs=in_specs,
        out_specs=pl.BlockSpec((7, 1, _H, _N, _TP),
                               lambda bi, j: (0, bi, 0, 0, j)),
        out_shape=jax.ShapeDtypeStruct((7, _B, _H, _N, _T), _F32),
        scratch_shapes=[pltpu.VMEM((1, _C), _F32)],
        compiler_params=pltpu.CompilerParams(
            dimension_semantics=("parallel", "arbitrary")),
        name="tmix_pre",
    )(*inputs)
    return outs


def _scan_kernel(d_ref, m_ref, y_ref, s_ref):
    j = pl.program_id(1)

    @pl.when(j == 0)
    def _():
        s_ref[...] = jnp.zeros_like(s_ref)

    tri = jnp.broadcast_to(m_ref[0], (_H, _L, _L))
    strict = m_ref[1]
    incl = m_ref[0]
    eye = m_ref[2]

    for i in range(_TC // _L):
        sl = slice(i * _L, (i + 1) * _L)
        r = d_ref[0, 0, :, :, sl]
        ew = d_ref[1, 0, :, :, sl]
        k = d_ref[2, 0, :, :, sl]
        v = d_ref[3, 0, :, :, sl]
        a = d_ref[4, 0, :, :, sl]
        b = d_ref[5, 0, :, :, sl]

        sm = _bdot(ew, tri, 2, 2)
        e1 = jnp.exp(sm)
        qn = 1.0 / e1
        ah = a * qn * jnp.exp(ew)
        bh = b * e1
        kh = k * e1
        rh = r * qn

        ahb, bhb, khb, rhb = _bf(ah), _bf(bh), _bf(kh), _bf(rh)
        vb = _bf(v)
        ab = _bdot(ahb, bhb, 1, 1) * strict
        akm = _bdot(ahb, khb, 1, 1) * strict
        rbm = _bdot(rhb, bhb, 1, 1) * incl
        rkm = _bdot(rhb, khb, 1, 1) * incl

        tm = eye + ab
        p = ab
        for _ in range(4):
            pb = _bf(p)
            p = _bdot(pb, pb, 2, 1)
            tm = tm + _bdot(_bf(tm), pb, 2, 1)

        st = s_ref[...]
        stb = _bf(st)
        ut = _bdot(stb, ahb, 1, 1) + _bdot(vb, _bf(akm), 2, 2)
        cct = _bdot(_bf(ut), _bf(tm), 2, 2)
        cctb = _bf(cct)
        yt = (_bdot(stb, rhb, 1, 1) + _bdot(cctb, _bf(rbm), 2, 2)
              + _bdot(vb, _bf(rkm), 2, 2))
        y_ref[0, :, :, sl] = yt

        qlc = qn[:, :, _L - 1:_L]
        s_ref[...] = (st + _bdot(bhb, cctb, 2, 2)
                      + _bdot(khb, vb, 2, 2)) * qlc


def _wkv7(pre):
    return pl.pallas_call(
        _scan_kernel,
        grid=(_B, _T // _TC),
        in_specs=[
            pl.BlockSpec((7, 1, _H, _N, _TC),
                         lambda bi, j: (0, bi, 0, 0, j)),
            pl.BlockSpec((3, _L, _L), lambda bi, j: (0, 0, 0)),
        ],
        out_specs=pl.BlockSpec((1, _H, _N, _TC),
                               lambda bi, j: (bi, 0, 0, j)),
        out_shape=jax.ShapeDtypeStruct((_B, _H, _N, _T), _F32),
        scratch_shapes=[pltpu.VMEM((_H, _N, _N), _F32)],
        compiler_params=pltpu.CompilerParams(
            dimension_semantics=("parallel", "arbitrary")),
        name="wkv7_scan",
    )(pre, _masks())


def _post_kernel(y_ref, p_ref, x_ref, vec_ref, wot_ref, hm_ref, o_ref):
    y = y_ref[0].reshape(_C, _TP)
    r = p_ref[0, 0].reshape(_C, _TP)
    k = p_ref[2, 0].reshape(_C, _TP)
    v = p_ref[3, 0].reshape(_C, _TP)
    g = p_ref[6, 0].reshape(_C, _TP)
    hm = hm_ref[...]
    m = _dot(hm, _bf(y)) * (1.0 / _N)
    c = y - m
    var = _dot(hm, _bf(c * c)) * (1.0 / _N)
    gn = (c * jax.lax.rsqrt(var + 0.00064) * vec_ref[:, 0:1]
          + vec_ref[:, 1:2])
    rk = _dot(hm, _bf(r * k * vec_ref[:, 2:3]))
    y2 = gn + rk * v
    o_ct = _dot(wot_ref[...], _bf(y2 * g))
    o_ref[0] = x_ref[0] + o_ct.T


def _tmix_post(y, pre, x, tp):
    vecs = jnp.stack([tp['gn_g'], tp['gn_b'], tp['r_k'].reshape(_C)], axis=1)
    act = pl.BlockSpec((1, _TP, _C), lambda bi, j: (bi, j, 0))
    actT = pl.BlockSpec((1, _H, _N, _TP), lambda bi, j: (bi, 0, 0, j))
    preT = pl.BlockSpec((7, 1, _H, _N, _TP), lambda bi, j: (0, bi, 0, 0, j))
    full = lambda s: pl.BlockSpec(s, lambda bi, j: tuple([0] * len(s)))
    return pl.pallas_call(
        _post_kernel,
        grid=(_B, _T // _TP),
        in_specs=[actT, preT, act, full((_C, 3)), full((_C, _C)),
                  full((_C, _C))],
        out_specs=act,
        out_shape=jax.ShapeDtypeStruct((_B, _T, _C), _F32),
        compiler_params=pltpu.CompilerParams(
            dimension_semantics=("parallel", "arbitrary")),
        name="tmix_post",
    )(y, pre, x, vecs, _bf(tp['Wo'].T), _bf(_hmask()))


def _ffn_kernel(x_ref, vec_ref, wk_ref, wv_ref, o_ref, prev_ref):
    j = pl.program_id(1)
    x = x_ref[0]
    xln = _ln(x, vec_ref[0], vec_ref[1])

    @pl.when(j == 0)
    def _():
        prev_ref[...] = jnp.zeros_like(prev_ref)

    xs = jnp.concatenate([prev_ref[...], xln[:_TP - 1]], axis=0)
    prev_ref[...] = xln[_TP - 1:_TP]
    xk = xln + (xs - xln) * vec_ref[2]
    h = jnp.square(jnp.maximum(_dot(_bf(xk), wk_ref[...]), 0.0))
    o_ref[0] = x + _dot(_bf(h), wv_ref[...])


def _ffn(x, fp, ln_g, ln_b):
    vecs = jnp.stack([ln_g, ln_b, fp['x_k']])
    act = pl.BlockSpec((1, _TP, _C), lambda bi, j: (bi, j, 0))
    full = lambda s: pl.BlockSpec(s, lambda bi, j: tuple([0] * len(s)))
    return pl.pallas_call(
        _ffn_kernel,
        grid=(_B, _T // _TP),
        in_specs=[act, full((3, _C)), full((_C, 4 * _C)), full((4 * _C, _C))],
        out_specs=act,
        out_shape=jax.ShapeDtypeStruct((_B, _T, _C), _F32),
        scratch_shapes=[pltpu.VMEM((1, _C), _F32)],
        compiler_params=pltpu.CompilerParams(
            dimension_semantics=("parallel", "arbitrary")),
        name="ffn",
    )(x, vecs, _bf(fp['Wk']), _bf(fp['Wv']))


def _head_kernel(x_ref, lnw_ref, w_ref, b_ref, o_ref):
    xln = _ln(x_ref[...], lnw_ref[0], lnw_ref[1])
    o_ref[...] = _dot(_bf(xln), w_ref[...]) + b_ref[...]


def _head(x, ln_g, ln_b, wout, bout):
    x2 = x.reshape(_B * _T, _C)
    lnw = jnp.stack([ln_g, ln_b])
    out = pl.pallas_call(
        _head_kernel,
        grid=(_V // _VT, (_B * _T) // _TR),
        in_specs=[
            pl.BlockSpec((_TR, _C), lambda jv, i: (i, 0)),
            pl.BlockSpec((2, _C), lambda jv, i: (0, 0)),
            pl.BlockSpec((_C, _VT), lambda jv, i: (0, jv)),
            pl.BlockSpec((1, _VT), lambda jv, i: (0, jv)),
        ],
        out_specs=pl.BlockSpec((_TR, _VT), lambda jv, i: (i, jv)),
        out_shape=jax.ShapeDtypeStruct((_B * _T, _V), _F32),
        compiler_params=pltpu.CompilerParams(
            dimension_semantics=("parallel", "arbitrary")),
        name="head_proj",
    )(x2, lnw, _bf(wout), bout.reshape(1, _V))
    return out.reshape(_B, _T, _V)


def _layer(x, tp, fp, lna_g, lna_b, lnb_g, lnb_b, v_first):
    pre = _tmix_pre(x, tp, lna_g, lna_b, v_first)
    y = _wkv7(pre)
    x = _tmix_post(y, pre, x, tp)
    x = _ffn(x, fp, lnb_g, lnb_b)
    return x, pre[3]


def kernel(tokens, params):
    p = params
    x = _embed(tokens, p['emb'])
    x, v_first = _layer(x, p['rwkv1'], p['ffn1'], p['ln1a_g'], p['ln1a_b'],
                        p['ln1b_g'], p['ln1b_b'], None)
    x, _ = _layer(x, p['rwkv2'], p['ffn2'], p['ln2a_g'], p['ln2a_b'],
                  p['ln2b_g'], p['ln2b_b'], v_first)
    return _head(x, p['lno_g'], p['lno_b'], p['Wout'], p['bout'])
